# baseline (speedup 1.0000x reference)
import jax
import jax.numpy as jnp
import numpy as np
from jax.experimental import pallas as pl
from jax.experimental.pallas import tpu as pltpu

_TOPK = 8
_BLOCK_TOKENS = 2048

_CONST_CACHE = {}


def _noise_eps_t(shape, dtype):
    key = ("epsT", shape, str(dtype))
    if key not in _CONST_CACHE:
        with jax.ensure_compile_time_eval():
            eps = jax.random.normal(jax.random.key(42), shape, dtype=dtype)
        _CONST_CACHE[key] = np.asarray(eps).T.copy()
    return _CONST_CACHE[key]


def _router_body(xt_ref, et_ref, out_ref, idx_ref):
    xt = xt_ref[...]
    n_experts = xt.shape[0]
    eps = et_ref[...]
    sp = jnp.maximum(xt, 0.0) + jnp.log1p(jnp.exp(-jnp.abs(xt)))
    orig = xt + eps * sp
    work = orig
    row_f = jax.lax.broadcasted_iota(jnp.int32, work.shape, 0).astype(
        jnp.float32)
    neg_inf = jnp.float32(-jnp.inf)
    idxs = []
    m0 = None
    for k in range(_TOPK):
        m = jnp.max(work, axis=0, keepdims=True)
        if k == 0:
            m0 = m
        a = jnp.min(jnp.where(work == m, row_f, float(n_experts)), axis=0,
                    keepdims=True)
        idxs.append(a)
        work = jnp.where(row_f == a, neg_inf, work)
    kept = work == neg_inf
    w = jnp.where(kept, jnp.exp(orig - m0), 0.0)
    total = jnp.sum(w, axis=0, keepdims=True)
    out_ref[...] = w * (1.0 / total)
    idx_ref[...] = jnp.concatenate(idxs, axis=0).astype(jnp.int32)


def kernel(logits):
    n_tokens, n_experts = logits.shape
    eps_t = _noise_eps_t(logits.shape, logits.dtype)
    block = min(_BLOCK_TOKENS, n_tokens)
    grid = n_tokens // block
    out_t, idx_t = pl.pallas_call(
        _router_body,
        grid=(grid,),
        in_specs=[
            pl.BlockSpec((n_experts, block), lambda i: (0, i)),
            pl.BlockSpec((n_experts, block), lambda i: (0, i)),
        ],
        out_specs=[
            pl.BlockSpec((n_experts, block), lambda i: (0, i)),
            pl.BlockSpec((_TOPK, block), lambda i: (0, i)),
        ],
        out_shape=[
            jax.ShapeDtypeStruct((n_experts, n_tokens), jnp.float32),
            jax.ShapeDtypeStruct((_TOPK, n_tokens), jnp.int32),
        ],
    )(logits.T, eps_t)
    return out_t.T, idx_t.T


# speedup vs baseline: 1.0147x; 1.0147x over previous
import jax
import jax.numpy as jnp
import numpy as np
from jax.experimental import pallas as pl
from jax.experimental.pallas import tpu as pltpu

_TOPK = 8
_BLOCK_TOKENS = 4096

_CONST_CACHE = {}


def _noise_eps_t(shape, dtype):
    key = ("epsT", shape, str(dtype))
    if key not in _CONST_CACHE:
        with jax.ensure_compile_time_eval():
            eps = jax.random.normal(jax.random.key(42), shape, dtype=dtype)
        _CONST_CACHE[key] = np.asarray(eps).T.copy()
    return _CONST_CACHE[key]


def _router_body(xt_ref, et_ref, out_ref, idx_ref):
    xt = xt_ref[...]
    n_experts = xt.shape[0]
    eps = et_ref[...]
    sp = jnp.maximum(xt, 0.0) + jnp.log1p(jnp.exp(-jnp.abs(xt)))
    orig = xt + eps * sp
    work = orig
    row_f = jax.lax.broadcasted_iota(jnp.int32, work.shape, 0).astype(
        jnp.float32)
    neg_inf = jnp.float32(-jnp.inf)
    idxs = []
    m0 = None
    for k in range(_TOPK):
        m = jnp.max(work, axis=0, keepdims=True)
        if k == 0:
            m0 = m
        a = jnp.min(jnp.where(work == m, row_f, float(n_experts)), axis=0,
                    keepdims=True)
        idxs.append(a)
        work = jnp.where(row_f == a, neg_inf, work)
    kept = work == neg_inf
    w = jnp.where(kept, jnp.exp(orig - m0), 0.0)
    total = jnp.sum(w, axis=0, keepdims=True)
    out_ref[...] = w * (1.0 / total)
    idx_ref[...] = jnp.concatenate(idxs, axis=0).astype(jnp.int32)


def kernel(logits):
    n_tokens, n_experts = logits.shape
    eps_t = _noise_eps_t(logits.shape, logits.dtype)
    block = min(_BLOCK_TOKENS, n_tokens)
    grid = n_tokens // block
    out_t, idx_t = pl.pallas_call(
        _router_body,
        grid=(grid,),
        in_specs=[
            pl.BlockSpec((n_experts, block), lambda i: (0, i)),
            pl.BlockSpec((n_experts, block), lambda i: (0, i)),
        ],
        out_specs=[
            pl.BlockSpec((n_experts, block), lambda i: (0, i)),
            pl.BlockSpec((_TOPK, block), lambda i: (0, i)),
        ],
        out_shape=[
            jax.ShapeDtypeStruct((n_experts, n_tokens), jnp.float32),
            jax.ShapeDtypeStruct((_TOPK, n_tokens), jnp.int32),
        ],
    )(logits.T, eps_t)
    return out_t.T, idx_t.T


# speedup vs baseline: 1.0147x; 1.0000x over previous
import jax
import jax.numpy as jnp
import numpy as np
from jax.experimental import pallas as pl

_TOPK = 8
_BLOCK_TOKENS = 4096

_CONST_CACHE = {}


def _noise_eps_t(shape, dtype):
    key = ("epsT", shape, str(dtype))
    if key not in _CONST_CACHE:
        with jax.ensure_compile_time_eval():
            eps = jax.random.normal(jax.random.key(42), shape, dtype=dtype)
        _CONST_CACHE[key] = np.asarray(eps).T.copy()
    return _CONST_CACHE[key]


def _router_body(xt_ref, et_ref, out_ref, idx_ref):
    xt = xt_ref[...]
    n_experts = xt.shape[0]
    eps = et_ref[...]
    sp = jnp.maximum(xt, 0.0) + jnp.log1p(jnp.exp(-jnp.abs(xt)))
    orig = xt + eps * sp
    work = orig
    row_f = jax.lax.broadcasted_iota(jnp.int32, work.shape, 0).astype(
        jnp.float32)
    neg_inf = jnp.float32(-jnp.inf)
    idxs = []
    m0 = None
    for k in range(_TOPK):
        m = jnp.max(work, axis=0, keepdims=True)
        if k == 0:
            m0 = m
        a = jnp.min(jnp.where(work == m, row_f, float(n_experts)), axis=0,
                    keepdims=True)
        idxs.append(a)
        work = jnp.where(row_f == a, neg_inf, work)
    kept = work == neg_inf
    w = jnp.where(kept, jnp.exp(orig - m0), 0.0)
    total = jnp.sum(w, axis=0, keepdims=True)
    out_ref[...] = w * (1.0 / total)
    idx_ref[...] = jnp.concatenate(idxs, axis=0).astype(jnp.int32)


def kernel(logits):
    n_tokens, n_experts = logits.shape
    eps_t = _noise_eps_t(logits.shape, logits.dtype)
    block = min(_BLOCK_TOKENS, n_tokens)
    grid = n_tokens // block
    out_t, idx_t = pl.pallas_call(
        _router_body,
        grid=(grid,),
        in_specs=[
            pl.BlockSpec((n_experts, block), lambda i: (0, i)),
            pl.BlockSpec((n_experts, block), lambda i: (0, i)),
        ],
        out_specs=[
            pl.BlockSpec((n_experts, block), lambda i: (0, i)),
            pl.BlockSpec((_TOPK, block), lambda i: (0, i)),
        ],
        out_shape=[
            jax.ShapeDtypeStruct((n_experts, n_tokens), jnp.float32),
            jax.ShapeDtypeStruct((_TOPK, n_tokens), jnp.int32),
        ],
    )(logits.T, eps_t)
    return out_t.T, idx_t.T
